# baseline (speedup 1.0000x reference)
import functools

import jax
import jax.numpy as jnp
from jax import lax
from jax.experimental import pallas as pl
from jax.experimental.pallas import tpu as pltpu
from jax.experimental.pallas import tpu_sc as plsc

NUM_CLASSES = 21
TOP_K = 64
B = 131072
D = 256
N_SEL = NUM_CLASSES * TOP_K
N_PAD = 2048


def _select_body(conf_ref, label_ref, out_ref, m_ref):
    conf = conf_ref[...]
    label = label_ref[...]
    neg_inf = jnp.finfo(jnp.float32).min
    out_iota = (lax.broadcasted_iota(jnp.int32, (16, 128), 0) * 128
                + lax.broadcasted_iota(jnp.int32, (16, 128), 1))
    acc0 = out_iota
    flat3 = (lax.broadcasted_iota(jnp.int32, (8, 128, 128), 0) * (128 * 128)
             + lax.broadcasted_iota(jnp.int32, (8, 128, 128), 1) * 128
             + lax.broadcasted_iota(jnp.int32, (8, 128, 128), 2))
    g_iota = lax.broadcasted_iota(jnp.int32, (8, 128), 0)
    i_iota = lax.broadcasted_iota(jnp.int32, (8, 128), 1)
    row_cols = lax.broadcasted_iota(jnp.int32, (1, 128), 1)
    big = jnp.int32(B)

    def class_body(c, acc):
        masked = jnp.where(label == c, conf, neg_inf)
        m_ref[...] = masked
        m3 = masked.reshape(8, 128, 128)
        rvk = jnp.max(m3, axis=2, keepdims=True)
        rb0 = jnp.min(jnp.where(m3 == rvk, flat3, big), axis=2)
        rv0 = rvk[:, :, 0]

        def pick_body(t, carry):
            rv, rb, acc = carry
            m = jnp.max(rv)
            bf = jnp.min(jnp.where(rv == m, rb, big))
            acc = jnp.where(out_iota == c * TOP_K + t, bf, acc)
            r = bf >> 7
            col = bf & 127
            row = jnp.where(row_cols == col, neg_inf, m_ref[pl.ds(r, 1), :])
            m_ref[pl.ds(r, 1), :] = row
            nrv = jnp.max(row)
            nrb = jnp.min(jnp.where(row == nrv, (r << 7) + row_cols, big))
            hit = (g_iota == (r >> 7)) & (i_iota == (r & 127))
            rv = jnp.where(hit, nrv, rv)
            rb = jnp.where(hit, nrb, rb)
            return rv, rb, acc

        _, _, acc = lax.fori_loop(0, TOP_K, pick_body, (rv0, rb0, acc))
        return acc

    out_ref[...] = lax.fori_loop(0, NUM_CLASSES, class_body, acc0)


def _tc_select(conf2, lab2, interpret=False):
    return pl.pallas_call(
        _select_body,
        out_shape=jax.ShapeDtypeStruct((16, 128), jnp.int32),
        scratch_shapes=[pltpu.VMEM((1024, 128), jnp.float32)],
        interpret=interpret,
    )(conf2, lab2)


def _sc_gather(idx_flat, table):
    info = plsc.get_sparse_core_info()
    nc, ns = info.num_cores, info.num_subcores
    nw = nc * ns
    per_w = N_PAD // nw
    mesh = plsc.VectorSubcoreMesh(core_axis_name="c", subcore_axis_name="s")

    @functools.partial(
        pl.kernel,
        mesh=mesh,
        out_type=jax.ShapeDtypeStruct((N_PAD, D), jnp.float32),
        scratch_types=[
            pltpu.VMEM((per_w,), jnp.int32),
            pltpu.VMEM((per_w, D), jnp.float32),
            pltpu.SemaphoreType.DMA,
        ],
    )
    def gather_k(idx_hbm, table_hbm, out_hbm, idx_v, rows_v, sem):
        wid = lax.axis_index("s") * nc + lax.axis_index("c")
        base = wid * per_w
        pltpu.sync_copy(idx_hbm.at[pl.ds(base, per_w)], idx_v)
        pltpu.async_copy(table_hbm.at[idx_v], rows_v, sem).wait()
        pltpu.sync_copy(rows_v, out_hbm.at[pl.ds(base, per_w)])

    return gather_k(idx_flat, table)


def kernel(confidence, label, contrast_feature):
    conf2 = confidence.reshape(1024, 128)
    lab2 = label.reshape(1024, 128)
    order = _tc_select(conf2, lab2).reshape(N_PAD)
    rows = _sc_gather(order, contrast_feature)
    return rows[:N_SEL].reshape(NUM_CLASSES, TOP_K, D)


# speedup vs baseline: 1.2081x; 1.2081x over previous
import functools

import jax
import jax.numpy as jnp
from jax import lax
from jax.experimental import pallas as pl
from jax.experimental.pallas import tpu as pltpu
from jax.experimental.pallas import tpu_sc as plsc

NUM_CLASSES = 21
TOP_K = 64
B = 131072
D = 256
N_SEL = NUM_CLASSES * TOP_K
N_PAD = 2048


def _select_body(conf_ref, label_ref, out_ref, *m_refs):
    conf = conf_ref[...]
    label = label_ref[...]
    neg_inf = jnp.finfo(jnp.float32).min
    out_iota = (lax.broadcasted_iota(jnp.int32, (16, 128), 0) * 128
                + lax.broadcasted_iota(jnp.int32, (16, 128), 1))
    acc0 = out_iota
    flat3 = (lax.broadcasted_iota(jnp.int32, (8, 128, 128), 0) * (128 * 128)
             + lax.broadcasted_iota(jnp.int32, (8, 128, 128), 1) * 128
             + lax.broadcasted_iota(jnp.int32, (8, 128, 128), 2))
    g_iota = lax.broadcasted_iota(jnp.int32, (8, 128), 0)
    i_iota = lax.broadcasted_iota(jnp.int32, (8, 128), 1)
    row_cols = lax.broadcasted_iota(jnp.int32, (1, 128), 1)
    big = jnp.int32(B)

    rvs, rbs = [], []
    for c in range(NUM_CLASSES):
        masked = jnp.where(label == c, conf, neg_inf)
        m_refs[c][...] = masked
        m3 = masked.reshape(8, 128, 128)
        rvk = jnp.max(m3, axis=2, keepdims=True)
        rbs.append(jnp.min(jnp.where(m3 == rvk, flat3, big), axis=2))
        rvs.append(rvk[:, :, 0])

    def pick_body(t, carry):
        rvs, rbs, acc = carry
        rvs, rbs = list(rvs), list(rbs)
        for c in range(NUM_CLASSES):
            rv, rb = rvs[c], rbs[c]
            m = jnp.max(rv)
            bf = jnp.min(jnp.where(rv == m, rb, big))
            acc = jnp.where(out_iota == c * TOP_K + t, bf, acc)
            r = bf >> 7
            col = bf & 127
            row = jnp.where(row_cols == col, neg_inf,
                            m_refs[c][pl.ds(r, 1), :])
            m_refs[c][pl.ds(r, 1), :] = row
            nrv = jnp.max(row)
            nrb = jnp.min(jnp.where(row == nrv, (r << 7) + row_cols, big))
            hit = (g_iota == (r >> 7)) & (i_iota == (r & 127))
            rvs[c] = jnp.where(hit, nrv, rv)
            rbs[c] = jnp.where(hit, nrb, rb)
        return tuple(rvs), tuple(rbs), acc

    _, _, acc = lax.fori_loop(0, TOP_K, pick_body, (tuple(rvs), tuple(rbs), acc0))
    out_ref[...] = acc


def _tc_select(conf2, lab2, interpret=False):
    return pl.pallas_call(
        _select_body,
        out_shape=jax.ShapeDtypeStruct((16, 128), jnp.int32),
        scratch_shapes=[pltpu.VMEM((1024, 128), jnp.float32)
                        for _ in range(NUM_CLASSES)],
        interpret=interpret,
    )(conf2, lab2)


def _sc_gather(idx_flat, table):
    info = plsc.get_sparse_core_info()
    nc, ns = info.num_cores, info.num_subcores
    nw = nc * ns
    per_w = N_PAD // nw
    mesh = plsc.VectorSubcoreMesh(core_axis_name="c", subcore_axis_name="s")

    @functools.partial(
        pl.kernel,
        mesh=mesh,
        out_type=jax.ShapeDtypeStruct((N_PAD, D), jnp.float32),
        scratch_types=[
            pltpu.VMEM((per_w,), jnp.int32),
            pltpu.VMEM((per_w, D), jnp.float32),
            pltpu.SemaphoreType.DMA,
        ],
    )
    def gather_k(idx_hbm, table_hbm, out_hbm, idx_v, rows_v, sem):
        wid = lax.axis_index("s") * nc + lax.axis_index("c")
        base = wid * per_w
        pltpu.sync_copy(idx_hbm.at[pl.ds(base, per_w)], idx_v)
        pltpu.async_copy(table_hbm.at[idx_v], rows_v, sem).wait()
        pltpu.sync_copy(rows_v, out_hbm.at[pl.ds(base, per_w)])

    return gather_k(idx_flat, table)


def kernel(confidence, label, contrast_feature):
    conf2 = confidence.reshape(1024, 128)
    lab2 = label.reshape(1024, 128)
    order = _tc_select(conf2, lab2).reshape(N_PAD)
    rows = _sc_gather(order, contrast_feature)
    return rows[:N_SEL].reshape(NUM_CLASSES, TOP_K, D)


# speedup vs baseline: 1.4364x; 1.1889x over previous
import functools

import jax
import jax.numpy as jnp
from jax import lax
from jax.experimental import pallas as pl
from jax.experimental.pallas import tpu as pltpu
from jax.experimental.pallas import tpu_sc as plsc

NUM_CLASSES = 21
TOP_K = 64
B = 131072
D = 256
N_SEL = NUM_CLASSES * TOP_K
N_PAD = 2048


def _select_body(conf_ref, label_ref, confT_ref, labelT_ref, out_ref):
    conf = conf_ref[...]
    label = label_ref[...]
    confT = confT_ref[...]
    labelT = labelT_ref[...]
    neg_inf = jnp.finfo(jnp.float32).min
    big = jnp.int32(B)
    out_iota = (lax.broadcasted_iota(jnp.int32, (16, 128), 0) * 128
                + lax.broadcasted_iota(jnp.int32, (16, 128), 1))
    lane128 = lax.broadcasted_iota(jnp.int32, (1, 128), 1)
    lane1024 = lax.broadcasted_iota(jnp.int32, (1, 1024), 1)
    sub64 = lax.broadcasted_iota(jnp.int32, (64, 1), 0)
    flatT = (lax.broadcasted_iota(jnp.int32, (128, 1024), 1) * 128
             + lax.broadcasted_iota(jnp.int32, (128, 1024), 0))

    rvs, rbs = [], []
    for c in range(NUM_CLASSES):
        mT = jnp.where(labelT == c, confT, neg_inf)
        rvk = jnp.max(mT, axis=0, keepdims=True)
        rbs.append(jnp.min(jnp.where(mT == rvk, flatT, big),
                           axis=0, keepdims=True))
        rvs.append(rvk)

    rows0 = tuple(jnp.zeros((64, 1), jnp.int32) for _ in range(NUM_CLASSES))

    def row_pick(t, carry):
        rvs, rows = carry
        rvs, rows = list(rvs), list(rows)
        for c in range(NUM_CLASSES):
            rv, rb = rvs[c], rbs[c]
            m = jnp.max(rv)
            bfr = jnp.min(jnp.where(rv == m, rb, big))
            rvs[c] = jnp.where(rb == bfr, neg_inf, rv)
            rows[c] = jnp.where(sub64 == t, bfr >> 7, rows[c])
        return tuple(rvs), tuple(rows)

    _, rows = lax.fori_loop(0, TOP_K, row_pick, (tuple(rvs), rows0))

    labf = label.astype(jnp.float32)
    ws, flats = [], []
    for c in range(NUM_CLASSES):
        r_c = rows[c]
        oh = jnp.where(r_c == lane1024, 1.0, 0.0)
        w = jnp.dot(oh, conf, precision=lax.Precision.HIGHEST,
                    preferred_element_type=jnp.float32)
        lg = jnp.dot(oh, labf, precision=lax.Precision.HIGHEST,
                     preferred_element_type=jnp.float32)
        ws.append(jnp.where(lg == jnp.float32(c), w, neg_inf))
        flats.append(r_c * 128 + lane128)

    def pick(t, carry):
        ws, acc = carry
        ws = list(ws)
        for c in range(NUM_CLASSES):
            w = ws[c]
            m = jnp.max(w)
            bf = jnp.min(jnp.where(w == m, flats[c], big))
            acc = jnp.where(out_iota == c * TOP_K + t, bf, acc)
            ws[c] = jnp.where(flats[c] == bf, neg_inf, w)
        return tuple(ws), acc

    _, acc = lax.fori_loop(0, TOP_K, pick, (tuple(ws), out_iota))
    out_ref[...] = acc


def _tc_select(conf2, lab2, interpret=False):
    return pl.pallas_call(
        _select_body,
        out_shape=jax.ShapeDtypeStruct((16, 128), jnp.int32),
        interpret=interpret,
    )(conf2, lab2, conf2.T, lab2.T)


def _sc_gather(idx_flat, table):
    info = plsc.get_sparse_core_info()
    nc, ns = info.num_cores, info.num_subcores
    nw = nc * ns
    per_w = N_PAD // nw
    mesh = plsc.VectorSubcoreMesh(core_axis_name="c", subcore_axis_name="s")

    @functools.partial(
        pl.kernel,
        mesh=mesh,
        out_type=jax.ShapeDtypeStruct((N_PAD, D), jnp.float32),
        scratch_types=[
            pltpu.VMEM((per_w,), jnp.int32),
            pltpu.VMEM((per_w, D), jnp.float32),
            pltpu.SemaphoreType.DMA,
        ],
    )
    def gather_k(idx_hbm, table_hbm, out_hbm, idx_v, rows_v, sem):
        wid = lax.axis_index("s") * nc + lax.axis_index("c")
        base = wid * per_w
        pltpu.sync_copy(idx_hbm.at[pl.ds(base, per_w)], idx_v)
        pltpu.async_copy(table_hbm.at[idx_v], rows_v, sem).wait()
        pltpu.sync_copy(rows_v, out_hbm.at[pl.ds(base, per_w)])

    return gather_k(idx_flat, table)


def kernel(confidence, label, contrast_feature):
    conf2 = confidence.reshape(1024, 128)
    lab2 = label.reshape(1024, 128)
    order = _tc_select(conf2, lab2).reshape(N_PAD)
    rows = _sc_gather(order, contrast_feature)
    return rows[:N_SEL].reshape(NUM_CLASSES, TOP_K, D)
